# Initial kernel scaffold; baseline (speedup 1.0000x reference)
import jax
import jax.numpy as jnp
from jax.experimental import pallas as pl


def kernel(edge_type, edge_index, x, bases, comp, root, bias):
    raise NotImplementedError("write your pallas kernel here")


# speedup vs baseline: 40.1401x; 40.1401x over previous
import functools

import jax
import jax.numpy as jnp
from jax import lax
from jax.experimental import pallas as pl
from jax.experimental.pallas import tpu as pltpu
from jax.experimental.pallas import tpu_sc as plsc

N_NODES = 10000
D = 128
N_REL = 12
N_BASES = 30
N_EDGES = 320000

PAD_KEYS = 120064
NC = 2
NS = 16
L = 16
CHUNK = 128
EDGES_PER_SC = N_EDGES // NC
CHUNKS_PER_SC = EDGES_PER_SC // CHUNK
MAX_CHUNKS_PER_TILE = -(-CHUNKS_PER_SC // NS)
KEY_SLICE = PAD_KEYS // NS
N_NODES_PAD = 10240
ROW_SLICE = N_NODES_PAD // NS
NB = 5
BLK = N_NODES // NB


def _sc_mesh():
    return plsc.VectorSubcoreMesh(core_axis_name="c", subcore_axis_name="s")


def _sc_hist(dst32, et32):
    @functools.partial(
        pl.kernel,
        mesh=_sc_mesh(),
        out_type=jax.ShapeDtypeStruct((NC * PAD_KEYS,), jnp.float32),
        scratch_types=[
            pltpu.VMEM_SHARED((PAD_KEYS,), jnp.float32),
            pltpu.VMEM((KEY_SLICE,), jnp.float32),
            pltpu.VMEM((CHUNK,), jnp.int32),
            pltpu.VMEM((CHUNK,), jnp.int32),
            pltpu.VMEM((CHUNK,), jnp.int32),
            pltpu.VMEM((CHUNK,), jnp.float32),
        ],
    )
    def hist(dst_hbm, et_hbm, out_hbm, cnt_sp, zbuf_v, dst_v, et_v, wkey_v,
             ones_v):
        c = lax.axis_index("c")
        s = lax.axis_index("s")

        @pl.loop(0, KEY_SLICE // L)
        def _(i):
            zbuf_v[pl.ds(i * L, L)] = jnp.full((L,), 0.0, jnp.float32)

        pltpu.sync_copy(zbuf_v, cnt_sp.at[pl.ds(s * KEY_SLICE, KEY_SLICE)])
        for j in range(CHUNK // L):
            ones_v[pl.ds(j * L, L)] = jnp.full((L,), 1.0, jnp.float32)
        plsc.subcore_barrier()

        @pl.loop(0, MAX_CHUNKS_PER_TILE)
        def _(k):
            ci = s + k * NS

            @pl.when(ci < CHUNKS_PER_SC)
            def _():
                e0 = c * EDGES_PER_SC + ci * CHUNK
                pltpu.sync_copy(dst_hbm.at[pl.ds(e0, CHUNK)], dst_v)
                pltpu.sync_copy(et_hbm.at[pl.ds(e0, CHUNK)], et_v)
                for j in range(CHUNK // L):
                    sl = pl.ds(j * L, L)
                    wkey_v[sl] = dst_v[sl] * N_REL + et_v[sl]
                pltpu.sync_copy(ones_v, cnt_sp.at[wkey_v], add=True)

        plsc.subcore_barrier()
        pltpu.sync_copy(cnt_sp.at[pl.ds(s * KEY_SLICE, KEY_SLICE)], zbuf_v)
        pltpu.sync_copy(
            zbuf_v,
            out_hbm.at[pl.ds(c * PAD_KEYS + s * KEY_SLICE, KEY_SLICE)])

    return hist(dst32, et32)


def _tc_weights(comp, bases2):
    def body(comp_ref, bases_ref, out_ref):
        out_ref[...] = jnp.dot(comp_ref[...], bases_ref[...],
                               preferred_element_type=jnp.float32)

    return pl.pallas_call(
        body,
        out_shape=jax.ShapeDtypeStruct((N_REL, D * D), jnp.float32),
    )(comp, bases2)


def _tc_transform(x, wall):
    def body(x_ref, w_ref, out_ref):
        out_ref[...] = jnp.dot(x_ref[...], w_ref[0],
                               preferred_element_type=jnp.float32)

    return pl.pallas_call(
        body,
        grid=(NB, N_REL),
        in_specs=[
            pl.BlockSpec((BLK, D), lambda b, r: (b, 0)),
            pl.BlockSpec((1, D, D), lambda b, r: (r, 0, 0)),
        ],
        out_specs=pl.BlockSpec((BLK, D), lambda b, r: (r * NB + b, 0)),
        out_shape=jax.ShapeDtypeStruct((N_REL * N_NODES, D), jnp.float32),
    )(x, wall)


def _tc_inv(cnt_part):
    def body(c_ref, out_ref):
        total = c_ref[0:1, :] + c_ref[1:2, :]
        out_ref[...] = jnp.where(total > 0.0,
                                 1.0 / jnp.maximum(total, 1.0), 0.0)

    return pl.pallas_call(
        body,
        out_shape=jax.ShapeDtypeStruct((1, PAD_KEYS), jnp.float32),
    )(cnt_part)


def _sc_scatter(src32, dst32, et32, h, inv1d):
    @functools.partial(
        pl.kernel,
        mesh=_sc_mesh(),
        out_type=jax.ShapeDtypeStruct((NC, N_NODES_PAD, D), jnp.float32),
        scratch_types=[
            pltpu.VMEM_SHARED((N_NODES_PAD, D), jnp.float32),
            pltpu.VMEM((CHUNK,), jnp.int32),
            pltpu.VMEM((CHUNK,), jnp.int32),
            pltpu.VMEM((CHUNK,), jnp.int32),
            pltpu.VMEM((CHUNK,), jnp.int32),
            pltpu.VMEM((CHUNK,), jnp.int32),
            pltpu.VMEM((CHUNK, D), jnp.float32),
            pltpu.VMEM((CHUNK,), jnp.float32),
            pltpu.SemaphoreType.DMA,
            pltpu.SemaphoreType.DMA,
        ],
    )
    def scatter(src_hbm, dst_hbm, et_hbm, h_hbm, inv_hbm, out_hbm,
                acc_sp, src_v, dst_v, et_v, gkey_v, wkey_v, rows_v, w_v,
                sem1, sem2):
        c = lax.axis_index("c")
        s = lax.axis_index("s")

        @pl.loop(0, CHUNK)
        def _(i):
            for j in range(D // L):
                rows_v[i, pl.ds(j * L, L)] = jnp.full((L,), 0.0, jnp.float32)

        for k in range(ROW_SLICE // CHUNK):
            pltpu.sync_copy(
                rows_v, acc_sp.at[pl.ds(s * ROW_SLICE + k * CHUNK, CHUNK)])
        plsc.subcore_barrier()

        @pl.loop(0, MAX_CHUNKS_PER_TILE)
        def _(k):
            ci = s + k * NS

            @pl.when(ci < CHUNKS_PER_SC)
            def _():
                e0 = c * EDGES_PER_SC + ci * CHUNK
                pltpu.sync_copy(src_hbm.at[pl.ds(e0, CHUNK)], src_v)
                pltpu.sync_copy(dst_hbm.at[pl.ds(e0, CHUNK)], dst_v)
                pltpu.sync_copy(et_hbm.at[pl.ds(e0, CHUNK)], et_v)
                for j in range(CHUNK // L):
                    sl = pl.ds(j * L, L)
                    gkey_v[sl] = et_v[sl] * N_NODES + src_v[sl]
                    wkey_v[sl] = dst_v[sl] * N_REL + et_v[sl]
                cp_rows = pltpu.async_copy(h_hbm.at[gkey_v], rows_v, sem1)
                cp_w = pltpu.async_copy(inv_hbm.at[wkey_v], w_v, sem2)
                cp_rows.wait()
                cp_w.wait()

                @pl.loop(0, CHUNK // L)
                def _(g):
                    i0 = g * L
                    wblk = w_v[pl.ds(i0, L)]
                    for e in range(L):
                        we = wblk[e]
                        for j in range(D // L):
                            sl2 = pl.ds(j * L, L)
                            rows_v[i0 + e, sl2] = rows_v[i0 + e, sl2] * we

                pltpu.sync_copy(rows_v, acc_sp.at[dst_v], add=True)

        plsc.subcore_barrier()
        for k in range(ROW_SLICE // CHUNK):
            off = s * ROW_SLICE + k * CHUNK
            pltpu.sync_copy(acc_sp.at[pl.ds(off, CHUNK)], rows_v)
            pltpu.sync_copy(
                rows_v,
                out_hbm.at[c, pl.ds(pl.multiple_of(off, CHUNK), CHUNK)])

    return scatter(src32, dst32, et32, h, inv1d)


def _tc_final(part, x, root, bias2d):
    def body(p_ref, x_ref, r_ref, b_ref, o_ref):
        acc = (p_ref[0] + p_ref[1]
               + jnp.dot(x_ref[...], r_ref[...],
                         preferred_element_type=jnp.float32)
               + b_ref[...])
        o_ref[...] = jnp.maximum(acc, 0.0)

    return pl.pallas_call(
        body,
        grid=(NB,),
        in_specs=[
            pl.BlockSpec((NC, BLK, D), lambda b: (0, b, 0)),
            pl.BlockSpec((BLK, D), lambda b: (b, 0)),
            pl.BlockSpec((D, D), lambda b: (0, 0)),
            pl.BlockSpec((1, D), lambda b: (0, 0)),
        ],
        out_specs=pl.BlockSpec((BLK, D), lambda b: (b, 0)),
        out_shape=jax.ShapeDtypeStruct((N_NODES, D), jnp.float32),
    )(part, x, root, bias2d)


def kernel(edge_type, edge_index, x, bases, comp, root, bias):
    et = edge_type.astype(jnp.int32)
    src = edge_index[0].astype(jnp.int32)
    dst = edge_index[1].astype(jnp.int32)

    wall = _tc_weights(comp, bases.reshape(N_BASES, D * D))
    h = _tc_transform(x, wall.reshape(N_REL, D, D))
    cnt = _sc_hist(dst, et).reshape(NC, PAD_KEYS)
    inv = _tc_inv(cnt).reshape(PAD_KEYS)
    part = _sc_scatter(src, dst, et, h, inv)
    return _tc_final(part, x, root, bias.reshape(1, D))
